# Initial kernel scaffold; baseline (speedup 1.0000x reference)
import jax
import jax.numpy as jnp
from jax.experimental import pallas as pl


def kernel(x, table):
    raise NotImplementedError("write your pallas kernel here")


# speedup vs baseline: 6.3300x; 6.3300x over previous
import jax
import jax.numpy as jnp
from jax.experimental import pallas as pl

_BLOCK_S = 256


def _bcast_kernel(t_ref, o_ref):
    n = o_ref.shape[1]
    o_ref[...] = jnp.broadcast_to(t_ref[...][:, None, :], (o_ref.shape[0], n, o_ref.shape[2]))


def kernel(x, table):
    S, N = x.shape
    _, E = table.shape
    out = pl.pallas_call(
        _bcast_kernel,
        grid=(S // _BLOCK_S,),
        in_specs=[pl.BlockSpec((_BLOCK_S, E), lambda i: (i, 0))],
        out_specs=pl.BlockSpec((_BLOCK_S, N, E), lambda i: (i, 0, 0)),
        out_shape=jax.ShapeDtypeStruct((S, N, E), table.dtype),
    )(table)
    return out
